# baseline (speedup 1.0000x reference)
import functools

import jax
import jax.numpy as jnp
from jax import lax
from jax.experimental import pallas as pl
from jax.experimental.pallas import tpu as pltpu
from jax.experimental.pallas import tpu_sc as plsc

NC = 2
NS = 16
NW = NC * NS
CC = 16
K = 128
NBUF = 1
IB = 4


def _leaky(x):
    return jnp.where(x >= 0, x, 0.01 * x)


def _combine_body(p0_ref, p1_ref, c0_ref, c1_ref, out_ref):
    cnt = c0_ref[:, 0:1] + c1_ref[:, 0:1]
    inv = jnp.where(cnt > 0, 1.0 / cnt, 0.0)
    out_ref[...] = (p0_ref[...] + p1_ref[...]) * inv


def _finish_body(nrows, p0_ref, p1_ref, c0_ref, c1_ref, whg_ref, bhg_ref,
                 wlin_ref, blin_ref, out_ref, acc_ref):
    c = p0_ref.shape[1]
    br = p0_ref.shape[0]
    i = pl.program_id(0)

    @pl.when(i == 0)
    def _():
        acc_ref[...] = jnp.zeros_like(acc_ref)

    cnt = c0_ref[:, 0:1] + c1_ref[:, 0:1]
    inv = jnp.where(cnt > 0, 1.0 / cnt, 0.0)
    s = (p0_ref[...] + p1_ref[...]) * inv
    o = lax.dot_general(s, whg_ref[...], (((1,), (1,)), ((), ())),
                        preferred_element_type=jnp.float32,
                        precision=lax.Precision.HIGHEST) + bhg_ref[...]
    y = _leaky(o)
    row = i * br + lax.broadcasted_iota(jnp.int32, (br, c), 0)
    y = jnp.where(row < nrows, y, 0.0)
    acc_ref[...] += lax.dot_general(y, y, (((0,), (0,)), ((), ())),
                                    preferred_element_type=jnp.float32,
                                    precision=lax.Precision.HIGHEST)

    @pl.when(i == pl.num_programs(0) - 1)
    def _():
        z = lax.dot_general(acc_ref[...], wlin_ref[...], (((1,), (1,)), ((), ())),
                            preferred_element_type=jnp.float32,
                            precision=lax.Precision.HIGHEST) + blin_ref[...]
        out_ref[...] = _leaky(z)


def _sc_pass_body(npad, c, chunks, nblocks,
                  gidx, sidx, table, onesrc, zrows, zcnt, out, outc,
                  gblk, sblk, rows, ones, acc, accc, gsem, ssem, csem, isem):
    ci = lax.axis_index("c")
    si = lax.axis_index("s")
    wid = si * NC + ci
    rps = npad // NS

    pltpu.sync_copy(onesrc, ones)

    pltpu.sync_copy(gidx.at[wid, pl.ds(0, IB)], gblk.at[0])
    pltpu.sync_copy(sidx.at[wid, pl.ds(0, IB)], sblk.at[0])

    pltpu.sync_copy(zrows.at[pl.ds(si * rps, rps)],
                    acc.at[pl.ds(si * rps, rps)])
    pltpu.sync_copy(zcnt.at[pl.ds(si * rps, rps)],
                    accc.at[pl.ds(si * rps, rps)])
    plsc.subcore_barrier()

    def body(j, carry):
        m = j // IB
        r = j - m * IB
        mslot = lax.rem(m, 2)
        pslot = lax.rem(m + 1, 2)

        @pl.when((r == 0) & (m > 0))
        def _():
            pltpu.make_async_copy(gidx.at[0, pl.ds(0, IB)],
                                  gblk.at[0], isem.at[mslot]).wait()
            pltpu.make_async_copy(sidx.at[0, pl.ds(0, IB)],
                                  sblk.at[0], isem.at[mslot]).wait()

        @pl.when((r == 0) & (m + 1 < nblocks))
        def _():
            pltpu.async_copy(gidx.at[wid, pl.ds((m + 1) * IB, IB)],
                             gblk.at[pslot], isem.at[pslot])
            pltpu.async_copy(sidx.at[wid, pl.ds((m + 1) * IB, IB)],
                             sblk.at[pslot], isem.at[pslot])

        pltpu.async_copy(table.at[gblk.at[mslot, r]], rows.at[0],
                         gsem.at[0])
        pltpu.make_async_copy(table.at[gblk.at[0, 0]], rows.at[0],
                              gsem.at[0]).wait()

        pltpu.async_copy(rows.at[0], acc.at[sblk.at[mslot, r]],
                         ssem.at[0], add=True)
        pltpu.async_copy(ones, accc.at[sblk.at[mslot, r]],
                         csem.at[0], add=True)
        pltpu.make_async_copy(rows.at[0], acc.at[gblk.at[0, 0]],
                              ssem.at[0]).wait()
        pltpu.make_async_copy(ones, accc.at[gblk.at[0, 0]],
                              csem.at[0]).wait()
        return carry

    lax.fori_loop(0, chunks, body, 0)

    plsc.subcore_barrier()
    pltpu.sync_copy(acc.at[pl.ds(si * rps, rps)],
                    out.at[ci, pl.ds(si * rps, rps)])
    pltpu.sync_copy(accc.at[pl.ds(si * rps, rps)],
                    outc.at[ci, pl.ds(si * rps, rps)])


@functools.cache
def _make_sc_pass(npad, c, e):
    ew = e // NW
    chunks = ew // K
    nblocks = chunks // IB
    assert ew % K == 0 and chunks % IB == 0 and npad % (8 * NS) == 0
    body = functools.partial(_sc_pass_body, npad, c, chunks, nblocks)
    return pl.kernel(
        body,
        out_type=(jax.ShapeDtypeStruct((NC, npad, c), jnp.float32),
                  jax.ShapeDtypeStruct((NC, npad, CC), jnp.float32)),
        mesh=plsc.VectorSubcoreMesh(core_axis_name="c", subcore_axis_name="s",
                                    num_cores=NC, num_subcores=NS),
        scratch_types=[
            pltpu.VMEM((2, IB, K), jnp.int32),
            pltpu.VMEM((2, IB, K), jnp.int32),
            pltpu.VMEM((2, K, c), jnp.float32),
            pltpu.VMEM((K, CC), jnp.float32),
            pltpu.VMEM_SHARED((npad, c), jnp.float32),
            pltpu.VMEM_SHARED((npad, CC), jnp.float32),
            pltpu.SemaphoreType.DMA((2,)),
            pltpu.SemaphoreType.DMA((1,)),
            pltpu.SemaphoreType.DMA((1,)),
            pltpu.SemaphoreType.DMA((2,)),
        ],
        compiler_params=pltpu.CompilerParams(use_tc_tiling_on_sc=False),
    )


def kernel(edge_index, emb, W_hg, b_hg, W_lin, b_lin):
    n, c = emb.shape
    e = edge_index.shape[1]
    brp = 1024
    npad = -(-n // brp) * brp
    assert npad % (8 * NS) == 0
    gridp = npad // brp
    ep = -(-e // (NW * K * IB)) * (NW * K * IB)
    ew = ep // NW

    pad = jnp.full((ep - e,), npad - 1, jnp.int32)
    node_idx = jnp.concatenate([edge_index[0], pad]).reshape(NW, ew // K, K)
    hedge_idx = jnp.concatenate([edge_index[1], pad]).reshape(NW, ew // K, K)
    emb_pad = jnp.zeros((npad, c), jnp.float32).at[:n].set(emb)
    zrows = jnp.zeros((npad, c), jnp.float32)
    zcnt = jnp.zeros((npad, CC), jnp.float32)
    onesrc = jnp.ones((K, CC), jnp.float32)

    sc_pass = _make_sc_pass(npad, c, ep)

    part1, cnt1 = sc_pass(node_idx, hedge_idx, emb_pad, onesrc, zrows, zcnt)

    ef = pl.pallas_call(
        _combine_body,
        grid=(gridp,),
        in_specs=[pl.BlockSpec((brp, c), lambda i: (i, 0)),
                  pl.BlockSpec((brp, c), lambda i: (i, 0)),
                  pl.BlockSpec((brp, CC), lambda i: (i, 0)),
                  pl.BlockSpec((brp, CC), lambda i: (i, 0))],
        out_specs=pl.BlockSpec((brp, c), lambda i: (i, 0)),
        out_shape=jax.ShapeDtypeStruct((npad, c), jnp.float32),
    )(part1[0], part1[1], cnt1[0], cnt1[1])

    part2, cnt2 = sc_pass(hedge_idx, node_idx, ef, onesrc, zrows, zcnt)

    out = pl.pallas_call(
        functools.partial(_finish_body, n),
        grid=(gridp,),
        in_specs=[pl.BlockSpec((brp, c), lambda i: (i, 0)),
                  pl.BlockSpec((brp, c), lambda i: (i, 0)),
                  pl.BlockSpec((brp, CC), lambda i: (i, 0)),
                  pl.BlockSpec((brp, CC), lambda i: (i, 0)),
                  pl.BlockSpec((c, c), lambda i: (0, 0)),
                  pl.BlockSpec((1, c), lambda i: (0, 0)),
                  pl.BlockSpec((c, c), lambda i: (0, 0)),
                  pl.BlockSpec((1, c), lambda i: (0, 0))],
        out_specs=pl.BlockSpec((c, c), lambda i: (0, 0)),
        out_shape=jax.ShapeDtypeStruct((c, c), jnp.float32),
        scratch_shapes=[pltpu.VMEM((c, c), jnp.float32)],
    )(part2[0], part2[1], cnt2[0], cnt2[1], W_hg, b_hg.reshape(1, c),
      W_lin, b_lin.reshape(1, c))

    return out


# speedup vs baseline: 1.4180x; 1.4180x over previous
import functools

import jax
import jax.numpy as jnp
from jax import lax
from jax.experimental import pallas as pl
from jax.experimental.pallas import tpu as pltpu
from jax.experimental.pallas import tpu_sc as plsc

NC = 2
NS = 16
NW = NC * NS
CC = 16
K = 128
NBUF = 1


def _leaky(x):
    return jnp.where(x >= 0, x, 0.01 * x)


def _combine_body(p0_ref, p1_ref, c0_ref, c1_ref, out_ref):
    cnt = c0_ref[:, 0:1] + c1_ref[:, 0:1]
    inv = jnp.where(cnt > 0, 1.0 / cnt, 0.0)
    out_ref[...] = (p0_ref[...] + p1_ref[...]) * inv


def _finish_body(nrows, p0_ref, p1_ref, c0_ref, c1_ref, whg_ref, bhg_ref,
                 wlin_ref, blin_ref, out_ref, acc_ref):
    c = p0_ref.shape[1]
    br = p0_ref.shape[0]
    i = pl.program_id(0)

    @pl.when(i == 0)
    def _():
        acc_ref[...] = jnp.zeros_like(acc_ref)

    cnt = c0_ref[:, 0:1] + c1_ref[:, 0:1]
    inv = jnp.where(cnt > 0, 1.0 / cnt, 0.0)
    s = (p0_ref[...] + p1_ref[...]) * inv
    o = lax.dot_general(s, whg_ref[...], (((1,), (1,)), ((), ())),
                        preferred_element_type=jnp.float32,
                        precision=lax.Precision.HIGHEST) + bhg_ref[...]
    y = _leaky(o)
    row = i * br + lax.broadcasted_iota(jnp.int32, (br, c), 0)
    y = jnp.where(row < nrows, y, 0.0)
    acc_ref[...] += lax.dot_general(y, y, (((0,), (0,)), ((), ())),
                                    preferred_element_type=jnp.float32,
                                    precision=lax.Precision.HIGHEST)

    @pl.when(i == pl.num_programs(0) - 1)
    def _():
        z = lax.dot_general(acc_ref[...], wlin_ref[...], (((1,), (1,)), ((), ())),
                            preferred_element_type=jnp.float32,
                            precision=lax.Precision.HIGHEST) + blin_ref[...]
        out_ref[...] = _leaky(z)


def _sc_pass_body(npad, c, nsuper,
                  gidx, sidx, table, onesrc, zrows, zcnt, out, outc,
                  gblk, sblk, rows, ones, acc, accc, gsem, ssem, csem, isem):
    ci = lax.axis_index("c")
    si = lax.axis_index("s")
    wid = si * NC + ci
    rps = npad // NS

    pltpu.sync_copy(onesrc, ones)

    pltpu.async_copy(gidx.at[wid, pl.ds(0, NBUF)], gblk.at[0], isem.at[0])
    pltpu.async_copy(sidx.at[wid, pl.ds(0, NBUF)], sblk.at[0], isem.at[0])

    pltpu.sync_copy(zrows.at[pl.ds(si * rps, rps)],
                    acc.at[pl.ds(si * rps, rps)])
    pltpu.sync_copy(zcnt.at[pl.ds(si * rps, rps)],
                    accc.at[pl.ds(si * rps, rps)])
    plsc.subcore_barrier()

    def body(j, carry):
        slot = lax.rem(j, 2)
        nxt = lax.rem(j + 1, 2)
        pltpu.make_async_copy(gidx.at[0, pl.ds(0, NBUF)], gblk.at[slot],
                              isem.at[slot]).wait()
        pltpu.make_async_copy(sidx.at[0, pl.ds(0, NBUF)], sblk.at[slot],
                              isem.at[slot]).wait()
        @pl.when(j + 1 < nsuper)
        def _():
            pltpu.async_copy(gidx.at[wid, pl.ds((j + 1) * NBUF, NBUF)],
                             gblk.at[nxt], isem.at[nxt])
            pltpu.async_copy(sidx.at[wid, pl.ds((j + 1) * NBUF, NBUF)],
                             sblk.at[nxt], isem.at[nxt])
        for b in range(NBUF):
            pltpu.async_copy(table.at[gblk.at[slot, b]], rows.at[b],
                             gsem.at[b])
        for b in range(NBUF):
            pltpu.make_async_copy(table.at[gblk.at[0, 0]], rows.at[b],
                                  gsem.at[b]).wait()
            pltpu.async_copy(rows.at[b], acc.at[sblk.at[slot, b]],
                             ssem.at[0], add=True)
            pltpu.async_copy(ones, accc.at[sblk.at[slot, b]],
                             csem.at[0], add=True)
            pltpu.make_async_copy(rows.at[b], acc.at[gblk.at[0, 0]],
                                  ssem.at[0]).wait()
            pltpu.make_async_copy(ones, accc.at[gblk.at[0, 0]],
                                  csem.at[0]).wait()
        return carry

    lax.fori_loop(0, nsuper, body, 0)

    plsc.subcore_barrier()
    pltpu.sync_copy(acc.at[pl.ds(si * rps, rps)],
                    out.at[ci, pl.ds(si * rps, rps)])
    pltpu.sync_copy(accc.at[pl.ds(si * rps, rps)],
                    outc.at[ci, pl.ds(si * rps, rps)])


@functools.cache
def _make_sc_pass(npad, c, e):
    ew = e // NW
    chunks = ew // K
    nsuper = chunks // NBUF
    assert ew % K == 0 and chunks % NBUF == 0 and npad % (8 * NS) == 0
    body = functools.partial(_sc_pass_body, npad, c, nsuper)
    return pl.kernel(
        body,
        out_type=(jax.ShapeDtypeStruct((NC, npad, c), jnp.float32),
                  jax.ShapeDtypeStruct((NC, npad, CC), jnp.float32)),
        mesh=plsc.VectorSubcoreMesh(core_axis_name="c", subcore_axis_name="s",
                                    num_cores=NC, num_subcores=NS),
        scratch_types=[
            pltpu.VMEM((2, NBUF, K), jnp.int32),
            pltpu.VMEM((2, NBUF, K), jnp.int32),
            pltpu.VMEM((NBUF, K, c), jnp.float32),
            pltpu.VMEM((K, CC), jnp.float32),
            pltpu.VMEM_SHARED((npad, c), jnp.float32),
            pltpu.VMEM_SHARED((npad, CC), jnp.float32),
            pltpu.SemaphoreType.DMA((NBUF,)),
            pltpu.SemaphoreType.DMA((NBUF,)),
            pltpu.SemaphoreType.DMA((NBUF,)),
            pltpu.SemaphoreType.DMA((2,)),
        ],
        compiler_params=pltpu.CompilerParams(use_tc_tiling_on_sc=False),
    )


def kernel(edge_index, emb, W_hg, b_hg, W_lin, b_lin):
    n, c = emb.shape
    e = edge_index.shape[1]
    brp = 1024
    npad = -(-n // brp) * brp
    assert npad % (8 * NS) == 0
    gridp = npad // brp
    ep = -(-e // (NW * K * NBUF)) * (NW * K * NBUF)
    ew = ep // NW

    pad = jnp.full((ep - e,), npad - 1, jnp.int32)
    node_idx = jnp.concatenate([edge_index[0], pad]).reshape(NW, ew // K, K)
    hedge_idx = jnp.concatenate([edge_index[1], pad]).reshape(NW, ew // K, K)
    emb_pad = jnp.zeros((npad, c), jnp.float32).at[:n].set(emb)
    zrows = jnp.zeros((npad, c), jnp.float32)
    zcnt = jnp.zeros((npad, CC), jnp.float32)
    onesrc = jnp.ones((K, CC), jnp.float32)

    sc_pass = _make_sc_pass(npad, c, ep)

    part1, cnt1 = sc_pass(node_idx, hedge_idx, emb_pad, onesrc, zrows, zcnt)

    ef = pl.pallas_call(
        _combine_body,
        grid=(gridp,),
        in_specs=[pl.BlockSpec((brp, c), lambda i: (i, 0)),
                  pl.BlockSpec((brp, c), lambda i: (i, 0)),
                  pl.BlockSpec((brp, CC), lambda i: (i, 0)),
                  pl.BlockSpec((brp, CC), lambda i: (i, 0))],
        out_specs=pl.BlockSpec((brp, c), lambda i: (i, 0)),
        out_shape=jax.ShapeDtypeStruct((npad, c), jnp.float32),
    )(part1[0], part1[1], cnt1[0], cnt1[1])

    part2, cnt2 = sc_pass(hedge_idx, node_idx, ef, onesrc, zrows, zcnt)

    out = pl.pallas_call(
        functools.partial(_finish_body, n),
        grid=(gridp,),
        in_specs=[pl.BlockSpec((brp, c), lambda i: (i, 0)),
                  pl.BlockSpec((brp, c), lambda i: (i, 0)),
                  pl.BlockSpec((brp, CC), lambda i: (i, 0)),
                  pl.BlockSpec((brp, CC), lambda i: (i, 0)),
                  pl.BlockSpec((c, c), lambda i: (0, 0)),
                  pl.BlockSpec((1, c), lambda i: (0, 0)),
                  pl.BlockSpec((c, c), lambda i: (0, 0)),
                  pl.BlockSpec((1, c), lambda i: (0, 0))],
        out_specs=pl.BlockSpec((c, c), lambda i: (0, 0)),
        out_shape=jax.ShapeDtypeStruct((c, c), jnp.float32),
        scratch_shapes=[pltpu.VMEM((c, c), jnp.float32)],
    )(part2[0], part2[1], cnt2[0], cnt2[1], W_hg, b_hg.reshape(1, c),
      W_lin, b_lin.reshape(1, c))

    return out
